# initial kernel scaffold (unmeasured)
import jax
import jax.numpy as jnp
from jax import lax
from jax.experimental import pallas as pl
from jax.experimental.pallas import tpu as pltpu


def kernel(
    x,
):
    def body(*refs):
        pass

    out_shape = jax.ShapeDtypeStruct(..., jnp.float32)
    return pl.pallas_call(body, out_shape=out_shape)(...)


# baseline (device time: 804120 ns/iter reference)
import jax
import jax.numpy as jnp
from jax import lax
from jax.experimental import pallas as pl
from jax.experimental.pallas import tpu as pltpu

M_PER = 4096
N_PER = 2048
N_OUT = 4096
K = 8
RC = M_PER // K


def kernel(x):
    def body(x_ref, out_ref, comm_x, comm_y, send_x, recv_x, send_y, recv_y):
        i = pl.program_id(0)
        mx = lax.axis_index("x")
        my = lax.axis_index("y")
        slot = lax.rem(i, 2)

        @pl.when(i == 0)
        def _():
            barrier = pltpu.get_barrier_semaphore()
            pl.semaphore_signal(
                barrier, inc=1, device_id=(1 - mx, my),
                device_id_type=pl.DeviceIdType.MESH,
            )
            pl.semaphore_signal(
                barrier, inc=1, device_id=(mx, 1 - my),
                device_id_type=pl.DeviceIdType.MESH,
            )
            pl.semaphore_wait(barrier, 2)

        rdma_x = pltpu.make_async_remote_copy(
            src_ref=x_ref,
            dst_ref=comm_x.at[slot],
            send_sem=send_x.at[slot],
            recv_sem=recv_x.at[slot],
            device_id=(1 - mx, my),
            device_id_type=pl.DeviceIdType.MESH,
        )
        rdma_x.start()
        rdma_x.wait()

        s = x_ref[...] + comm_x[slot]
        col = my * N_PER
        out_ref[:, pl.ds(col, N_PER)] = s

        rdma_y = pltpu.make_async_remote_copy(
            src_ref=out_ref.at[:, pl.ds(col, N_PER)],
            dst_ref=comm_y.at[slot],
            send_sem=send_y.at[slot],
            recv_sem=recv_y.at[slot],
            device_id=(mx, 1 - my),
            device_id_type=pl.DeviceIdType.MESH,
        )
        rdma_y.start()
        rdma_y.wait()

        out_ref[:, pl.ds((1 - my) * N_PER, N_PER)] = comm_y[slot]

    return pl.pallas_call(
        body,
        grid=(K,),
        out_shape=jax.ShapeDtypeStruct((M_PER, N_OUT), jnp.float32),
        in_specs=[pl.BlockSpec((RC, N_PER), lambda i: (i, 0))],
        out_specs=pl.BlockSpec((RC, N_OUT), lambda i: (i, 0)),
        scratch_shapes=[
            pltpu.VMEM((2, RC, N_PER), jnp.float32),
            pltpu.VMEM((2, RC, N_PER), jnp.float32),
            pltpu.SemaphoreType.DMA((2,)),
            pltpu.SemaphoreType.DMA((2,)),
            pltpu.SemaphoreType.DMA((2,)),
            pltpu.SemaphoreType.DMA((2,)),
        ],
        compiler_params=pltpu.CompilerParams(
            collective_id=0,
            dimension_semantics=("arbitrary",),
            vmem_limit_bytes=64 * 1024 * 1024,
        ),
    )(x)


# device time: 481637 ns/iter; 1.6696x vs baseline; 1.6696x over previous
import jax
import jax.numpy as jnp
from jax import lax
from jax.experimental import pallas as pl
from jax.experimental.pallas import tpu as pltpu

M_PER = 4096
N_PER = 2048
N_OUT = 4096
K = 8
RC = M_PER // K


def kernel(x):
    def body(x_blk, x_any, out_ref, comm_x, comm_y,
             send_x, recv_x, send_y, recv_y):
        i = pl.program_id(0)
        mx = lax.axis_index("x")
        my = lax.axis_index("y")
        slot = lax.rem(i, 2)
        nslot = lax.rem(i + 1, 2)

        def x_rdma(j, s):
            return pltpu.make_async_remote_copy(
                src_ref=x_any.at[pl.ds(j * RC, RC), :],
                dst_ref=comm_x.at[s],
                send_sem=send_x.at[s],
                recv_sem=recv_x.at[s],
                device_id=(1 - mx, my),
                device_id_type=pl.DeviceIdType.MESH,
            )

        @pl.when(i == 0)
        def _():
            barrier = pltpu.get_barrier_semaphore()
            pl.semaphore_signal(
                barrier, inc=1, device_id=(1 - mx, my),
                device_id_type=pl.DeviceIdType.MESH,
            )
            pl.semaphore_signal(
                barrier, inc=1, device_id=(mx, 1 - my),
                device_id_type=pl.DeviceIdType.MESH,
            )
            pl.semaphore_wait(barrier, 2)
            x_rdma(i, slot).start()

        x_rdma(i, slot).wait()

        @pl.when(i + 1 < K)
        def _():
            x_rdma(i + 1, nslot).start()

        s = x_blk[...] + comm_x[slot]
        col = my * N_PER
        out_ref[:, pl.ds(col, N_PER)] = s

        rdma_y = pltpu.make_async_remote_copy(
            src_ref=out_ref.at[:, pl.ds(col, N_PER)],
            dst_ref=comm_y.at[slot],
            send_sem=send_y.at[slot],
            recv_sem=recv_y.at[slot],
            device_id=(mx, 1 - my),
            device_id_type=pl.DeviceIdType.MESH,
        )
        rdma_y.start()
        rdma_y.wait()

        out_ref[:, pl.ds((1 - my) * N_PER, N_PER)] = comm_y[slot]

    return pl.pallas_call(
        body,
        grid=(K,),
        out_shape=jax.ShapeDtypeStruct((M_PER, N_OUT), jnp.float32),
        in_specs=[
            pl.BlockSpec((RC, N_PER), lambda i: (i, 0)),
            pl.BlockSpec(memory_space=pl.ANY),
        ],
        out_specs=pl.BlockSpec((RC, N_OUT), lambda i: (i, 0)),
        scratch_shapes=[
            pltpu.VMEM((2, RC, N_PER), jnp.float32),
            pltpu.VMEM((2, RC, N_PER), jnp.float32),
            pltpu.SemaphoreType.DMA((2,)),
            pltpu.SemaphoreType.DMA((2,)),
            pltpu.SemaphoreType.DMA((2,)),
            pltpu.SemaphoreType.DMA((2,)),
        ],
        compiler_params=pltpu.CompilerParams(
            collective_id=0,
            dimension_semantics=("arbitrary",),
            vmem_limit_bytes=64 * 1024 * 1024,
        ),
    )(x, x)


# device time: 434390 ns/iter; 1.8511x vs baseline; 1.1088x over previous
import jax
import jax.numpy as jnp
from jax import lax
from jax.experimental import pallas as pl
from jax.experimental.pallas import tpu as pltpu

M_PER = 4096
N_PER = 2048
N_OUT = 4096
K = 16
RC = M_PER // K
XS = 4


def kernel(x):
    def body(x_blk, x_any, out_any, comm_x, sbuf,
             send_x, recv_x, send_y, recv_y, cp_sem):
        i = pl.program_id(0)
        mx = lax.axis_index("x")
        my = lax.axis_index("y")
        s2 = lax.rem(i, 2)
        n2 = lax.rem(i + 1, 2)
        col = my * N_PER

        def x_rdma(j):
            return pltpu.make_async_remote_copy(
                src_ref=x_any.at[pl.ds(j * RC, RC), :],
                dst_ref=comm_x.at[lax.rem(j, XS)],
                send_sem=send_x.at[lax.rem(j, XS)],
                recv_sem=recv_x.at[lax.rem(j, XS)],
                device_id=(1 - mx, my),
                device_id_type=pl.DeviceIdType.MESH,
            )

        def y_rdma(j, s):
            return pltpu.make_async_remote_copy(
                src_ref=sbuf.at[s],
                dst_ref=out_any.at[pl.ds(j * RC, RC), pl.ds(col, N_PER)],
                send_sem=send_y.at[s],
                recv_sem=recv_y.at[j],
                device_id=(mx, 1 - my),
                device_id_type=pl.DeviceIdType.MESH,
            )

        def cp_own(j, s):
            return pltpu.make_async_copy(
                sbuf.at[s],
                out_any.at[pl.ds(j * RC, RC), pl.ds(col, N_PER)],
                cp_sem.at[s],
            )

        @pl.when(i == 0)
        def _():
            barrier = pltpu.get_barrier_semaphore()
            pl.semaphore_signal(
                barrier, inc=1, device_id=(1 - mx, my),
                device_id_type=pl.DeviceIdType.MESH,
            )
            pl.semaphore_signal(
                barrier, inc=1, device_id=(mx, 1 - my),
                device_id_type=pl.DeviceIdType.MESH,
            )
            pl.semaphore_wait(barrier, 2)
            x_rdma(0).start()
            x_rdma(1).start()

        x_rdma(i).wait()

        @pl.when(i < K - 2)
        def _():
            x_rdma(i + 2).start()

        @pl.when(i >= 2)
        def _():
            y_rdma(i - 2, s2).wait_send()
            cp_own(i - 2, s2).wait()

        sbuf[s2] = x_blk[...] + comm_x[lax.rem(i, XS)]

        cp_own(i, s2).start()
        y_rdma(i, s2).start()

        @pl.when(i >= 1)
        def _():
            y_rdma(i - 1, n2).wait_recv()

        @pl.when(i == K - 1)
        def _():
            y_rdma(i, s2).wait_recv()
            y_rdma(i, s2).wait_send()
            y_rdma(i - 1, n2).wait_send()
            cp_own(i, s2).wait()
            cp_own(i - 1, n2).wait()

    return pl.pallas_call(
        body,
        grid=(K,),
        out_shape=jax.ShapeDtypeStruct((M_PER, N_OUT), jnp.float32),
        in_specs=[
            pl.BlockSpec((RC, N_PER), lambda i: (i, 0)),
            pl.BlockSpec(memory_space=pl.ANY),
        ],
        out_specs=pl.BlockSpec(memory_space=pl.ANY),
        scratch_shapes=[
            pltpu.VMEM((XS, RC, N_PER), jnp.float32),
            pltpu.VMEM((2, RC, N_PER), jnp.float32),
            pltpu.SemaphoreType.DMA((XS,)),
            pltpu.SemaphoreType.DMA((XS,)),
            pltpu.SemaphoreType.DMA((2,)),
            pltpu.SemaphoreType.DMA((K,)),
            pltpu.SemaphoreType.DMA((2,)),
        ],
        compiler_params=pltpu.CompilerParams(
            collective_id=0,
            dimension_semantics=("arbitrary",),
            vmem_limit_bytes=64 * 1024 * 1024,
        ),
    )(x, x)
